# Initial kernel scaffold; baseline (speedup 1.0000x reference)
import jax
import jax.numpy as jnp
from jax.experimental import pallas as pl


def kernel(v, n_feats, aud, n_auds, ocr, n_ocrs, o, n_answ, temp_vid, temp_aud, temp_ocr):
    raise NotImplementedError("write your pallas kernel here")


# speedup vs baseline: 3.0518x; 3.0518x over previous
import jax
import jax.numpy as jnp
from jax.experimental import pallas as pl
from jax.experimental.pallas import tpu as pltpu

_TOPK = 25
_BLK = 2048
_NSEL_PAD = 256
_DUMP = 255


def _score_kernel(v_ref, nf_ref, e_ref, et_ref):
    v = v_ref[...]
    qn = v / jnp.maximum(jnp.sqrt(jnp.sum(v * v, axis=1, keepdims=True)), 1e-12)
    k = nf_ref[...]
    kn = k / jnp.maximum(jnp.sqrt(jnp.sum(k * k, axis=1, keepdims=True)), 1e-12)
    s = jax.lax.dot_general(qn, kn, (((1,), (1,)), ((), ())),
                            preferred_element_type=jnp.float32)
    e_ref[...] = jnp.exp(jnp.clip(s, 0.0, 1.0))
    st = jax.lax.dot_general(kn, qn, (((1,), (1,)), ((), ())),
                             preferred_element_type=jnp.float32)
    et_ref[...] = jnp.exp(jnp.clip(st, 0.0, 1.0))


def _topk_kernel(e_ref, idx_ref, work_ref):
    work_ref[...] = e_ref[...]
    nb, n = work_ref.shape
    iota = jax.lax.broadcasted_iota(jnp.int32, (nb, n), 1)

    for t in range(_TOPK):
        x = work_ref[...]
        m = jnp.max(x, axis=1, keepdims=True)
        am = jnp.min(jnp.where(x == m, iota, n), axis=1, keepdims=True)
        idx_ref[:, t:t + 1] = am
        work_ref[...] = jnp.where(iota == am, -1.0, x)


def _gather_kernel(idx_ref, tv_ref, et_ref, o0_ref, o1_ref, o2_ref,
                   na_ref, out_ref, wt_ref, rows_ref, seen_ref, wsem, rsem):
    wt_ref[...] = jnp.zeros_like(wt_ref)
    rows_ref[...] = jnp.zeros_like(rows_ref)

    def clear_body(i, carry):
        seen_ref[i] = 0
        return carry

    jax.lax.fori_loop(0, seen_ref.shape[0], clear_body, 0)
    nb, ksel = idx_ref.shape

    def start_body(j, carry):
        b = j // ksel
        t = j - b * ksel
        d = idx_ref[b, t]
        word = d // 32
        bit = d - word * 32
        seen = seen_ref[word]
        dup = (seen >> bit) & 1
        seen_ref[word] = seen | (1 << bit)
        je = jnp.where(dup == 1, _DUMP, j)
        pltpu.make_async_copy(et_ref.at[pl.ds(d, 1), :],
                              wt_ref.at[pl.ds(je, 1), :], wsem).start()
        pltpu.make_async_copy(na_ref.at[pl.ds(d, 1), :],
                              rows_ref.at[pl.ds(je, 1), :], rsem).start()
        return carry

    jax.lax.fori_loop(0, nb * ksel, start_body, 0)

    def wait_body(j, carry):
        pltpu.make_async_copy(et_ref.at[pl.ds(0, 1), :],
                              wt_ref.at[pl.ds(0, 1), :], wsem).wait()
        pltpu.make_async_copy(na_ref.at[pl.ds(0, 1), :],
                              rows_ref.at[pl.ds(0, 1), :], rsem).wait()
        return carry

    jax.lax.fori_loop(0, nb * ksel, wait_body, 0)

    wt_ref[_DUMP:_DUMP + 1, :] = jnp.zeros((1, nb), jnp.float32)
    denom_t = jnp.sum(et_ref[...], axis=0, keepdims=True)
    gate = 2.0 * jax.nn.sigmoid(tv_ref[0])
    wt = wt_ref[...] * (gate / denom_t)
    oia = jax.lax.dot_general(wt, rows_ref[...], (((0,), (0,)), ((), ())),
                              preferred_element_type=jnp.float32)
    out_ref[:, 0:1] = jnp.sum(o0_ref[...] * oia, axis=1, keepdims=True)
    out_ref[:, 1:2] = jnp.sum(o1_ref[...] * oia, axis=1, keepdims=True)
    out_ref[:, 2:3] = jnp.sum(o2_ref[...] * oia, axis=1, keepdims=True)


def kernel(v, n_feats, aud, n_auds, ocr, n_ocrs, o, n_answ, temp_vid,
           temp_aud, temp_ocr):
    del aud, n_auds, ocr, n_ocrs, temp_aud, temp_ocr
    bq, d = v.shape
    n = n_feats.shape[0]

    e, et = pl.pallas_call(
        _score_kernel,
        grid=(n // _BLK,),
        in_specs=[pl.BlockSpec((bq, d), lambda i: (0, 0)),
                  pl.BlockSpec((_BLK, d), lambda i: (i, 0))],
        out_specs=(pl.BlockSpec((bq, _BLK), lambda i: (0, i)),
                   pl.BlockSpec((_BLK, bq), lambda i: (i, 0))),
        out_shape=(jax.ShapeDtypeStruct((bq, n), jnp.float32),
                   jax.ShapeDtypeStruct((n, bq), jnp.float32)),
    )(v, n_feats)

    idx = pl.pallas_call(
        _topk_kernel,
        out_shape=jax.ShapeDtypeStruct((bq, _TOPK), jnp.int32),
        scratch_shapes=[pltpu.VMEM((bq, n), jnp.float32)],
    )(e)

    o0, o1, o2 = o[:, 0, :], o[:, 1, :], o[:, 2, :]
    scores = pl.pallas_call(
        _gather_kernel,
        in_specs=[
            pl.BlockSpec(memory_space=pltpu.SMEM),
            pl.BlockSpec(memory_space=pltpu.SMEM),
            pl.BlockSpec(memory_space=pltpu.VMEM),
            pl.BlockSpec(memory_space=pltpu.VMEM),
            pl.BlockSpec(memory_space=pltpu.VMEM),
            pl.BlockSpec(memory_space=pltpu.VMEM),
            pl.BlockSpec(memory_space=pl.ANY),
        ],
        out_shape=jax.ShapeDtypeStruct((bq, 3), jnp.float32),
        scratch_shapes=[pltpu.VMEM((_NSEL_PAD, bq), jnp.float32),
                        pltpu.VMEM((_NSEL_PAD, n_answ.shape[1]), jnp.float32),
                        pltpu.SMEM((n // 32,), jnp.int32),
                        pltpu.SemaphoreType.DMA,
                        pltpu.SemaphoreType.DMA],
    )(idx, temp_vid, et, o0, o1, o2, n_answ)
    return scores


# speedup vs baseline: 3.1892x; 1.0450x over previous
import jax
import jax.numpy as jnp
from jax.experimental import pallas as pl
from jax.experimental.pallas import tpu as pltpu

_TOPK = 25
_BLK = 2048
_NSEL_PAD = 256
_DUMP = 255


def _score_kernel(v_ref, nf_ref, e_ref, et_ref):
    v = v_ref[...]
    qn = v / jnp.maximum(jnp.sqrt(jnp.sum(v * v, axis=1, keepdims=True)), 1e-12)
    k = nf_ref[...]
    kn = k / jnp.maximum(jnp.sqrt(jnp.sum(k * k, axis=1, keepdims=True)), 1e-12)
    s = jax.lax.dot_general(qn, kn, (((1,), (1,)), ((), ())),
                            preferred_element_type=jnp.float32)
    e_ref[...] = jnp.exp(jnp.clip(s, 0.0, 1.0))
    st = jax.lax.dot_general(kn, qn, (((1,), (1,)), ((), ())),
                             preferred_element_type=jnp.float32)
    et_ref[...] = jnp.exp(jnp.clip(st, 0.0, 1.0))


def _topk_kernel(e_ref, idx_ref, den_ref, work_ref):
    work_ref[...] = e_ref[...]
    nb, n = work_ref.shape
    den_ref[...] = jnp.sum(e_ref[...], axis=1, keepdims=True)
    iota = jax.lax.broadcasted_iota(jnp.int32, (nb, n), 1)

    for t in range(_TOPK):
        x = work_ref[...]
        m = jnp.max(x, axis=1, keepdims=True)
        am = jnp.min(jnp.where(x == m, iota, n), axis=1, keepdims=True)
        idx_ref[:, t:t + 1] = am
        work_ref[...] = jnp.where(iota == am, -1.0, x)


def _gather_kernel(idx_ref, tv_ref, den_ref, et_ref, o0_ref, o1_ref, o2_ref,
                   na_ref, out_ref, wt_ref, rows_ref, seen_ref, wsem, rsem):
    wt_ref[...] = jnp.zeros_like(wt_ref)
    rows_ref[...] = jnp.zeros_like(rows_ref)

    def clear_body(i, carry):
        seen_ref[i] = 0
        return carry

    jax.lax.fori_loop(0, seen_ref.shape[0], clear_body, 0)
    nb, ksel = idx_ref.shape

    def start_body(j, carry):
        b = j // ksel
        t = j - b * ksel
        d = idx_ref[b, t]
        word = d // 32
        bit = d - word * 32
        seen = seen_ref[word]
        dup = (seen >> bit) & 1
        seen_ref[word] = seen | (1 << bit)
        je = jnp.where(dup == 1, _DUMP, j)
        pltpu.make_async_copy(et_ref.at[pl.ds(d, 1), :],
                              wt_ref.at[pl.ds(je, 1), :], wsem).start()
        pltpu.make_async_copy(na_ref.at[pl.ds(d, 1), :],
                              rows_ref.at[pl.ds(je, 1), :], rsem).start()
        return carry

    jax.lax.fori_loop(0, nb * ksel, start_body, 0)

    def wait_body(j, carry):
        pltpu.make_async_copy(et_ref.at[pl.ds(0, 1), :],
                              wt_ref.at[pl.ds(0, 1), :], wsem).wait()
        pltpu.make_async_copy(na_ref.at[pl.ds(0, 1), :],
                              rows_ref.at[pl.ds(0, 1), :], rsem).wait()
        return carry

    jax.lax.fori_loop(0, nb * ksel, wait_body, 0)

    wt_ref[_DUMP:_DUMP + 1, :] = jnp.zeros((1, nb), jnp.float32)
    gate = 2.0 * jax.nn.sigmoid(tv_ref[0])
    lane = jax.lax.broadcasted_iota(jnp.int32, (1, nb), 1)
    sv = jnp.zeros((1, nb), jnp.float32)
    for b in range(nb):
        sv = jnp.where(lane == b, gate / den_ref[b, 0], sv)
    wt = wt_ref[...] * sv
    oia = jax.lax.dot_general(wt, rows_ref[...], (((0,), (0,)), ((), ())),
                              preferred_element_type=jnp.float32)
    out_ref[:, 0:1] = jnp.sum(o0_ref[...] * oia, axis=1, keepdims=True)
    out_ref[:, 1:2] = jnp.sum(o1_ref[...] * oia, axis=1, keepdims=True)
    out_ref[:, 2:3] = jnp.sum(o2_ref[...] * oia, axis=1, keepdims=True)


def kernel(v, n_feats, aud, n_auds, ocr, n_ocrs, o, n_answ, temp_vid,
           temp_aud, temp_ocr):
    del aud, n_auds, ocr, n_ocrs, temp_aud, temp_ocr
    bq, d = v.shape
    n = n_feats.shape[0]

    e, et = pl.pallas_call(
        _score_kernel,
        grid=(n // _BLK,),
        in_specs=[pl.BlockSpec((bq, d), lambda i: (0, 0)),
                  pl.BlockSpec((_BLK, d), lambda i: (i, 0))],
        out_specs=(pl.BlockSpec((bq, _BLK), lambda i: (0, i)),
                   pl.BlockSpec((_BLK, bq), lambda i: (i, 0))),
        out_shape=(jax.ShapeDtypeStruct((bq, n), jnp.float32),
                   jax.ShapeDtypeStruct((n, bq), jnp.float32)),
    )(v, n_feats)

    idx, den = pl.pallas_call(
        _topk_kernel,
        out_shape=(jax.ShapeDtypeStruct((bq, _TOPK), jnp.int32),
                   jax.ShapeDtypeStruct((bq, 1), jnp.float32)),
        scratch_shapes=[pltpu.VMEM((bq, n), jnp.float32)],
    )(e)

    o0, o1, o2 = o[:, 0, :], o[:, 1, :], o[:, 2, :]
    scores = pl.pallas_call(
        _gather_kernel,
        in_specs=[
            pl.BlockSpec(memory_space=pltpu.SMEM),
            pl.BlockSpec(memory_space=pltpu.SMEM),
            pl.BlockSpec(memory_space=pltpu.SMEM),
            pl.BlockSpec(memory_space=pltpu.VMEM),
            pl.BlockSpec(memory_space=pltpu.VMEM),
            pl.BlockSpec(memory_space=pltpu.VMEM),
            pl.BlockSpec(memory_space=pltpu.VMEM),
            pl.BlockSpec(memory_space=pl.ANY),
        ],
        out_shape=jax.ShapeDtypeStruct((bq, 3), jnp.float32),
        scratch_shapes=[pltpu.VMEM((_NSEL_PAD, bq), jnp.float32),
                        pltpu.VMEM((_NSEL_PAD, n_answ.shape[1]), jnp.float32),
                        pltpu.SMEM((n // 32,), jnp.int32),
                        pltpu.SemaphoreType.DMA,
                        pltpu.SemaphoreType.DMA],
    )(idx, temp_vid, den, et, o0, o1, o2, n_answ)
    return scores
